# baseline (speedup 1.0000x reference)
import functools

import jax
import jax.numpy as jnp
from jax import lax
from jax.experimental import pallas as pl
from jax.experimental.pallas import tpu as pltpu
from jax.experimental.pallas import tpu_sc as plsc

D = 2048
E = 8
K = 2
H = 1024
N = 2048
TM = 256
NT_R = N * K // TM + E
NT = NT_R + N // TM
SHBASE = NT_R * TM
PTOT = NT * TM
HB = 256
CB = 256

_HIGH = jax.lax.Precision.HIGHEST


def _router_kernel(lg_ref, dest1_ref, dest2_ref, w1_ref, w2_ref,
                   aux_ref, te_ref):
    logits = lg_ref[...]

    lane = lax.broadcasted_iota(jnp.int32, (N, E), 1)
    m1 = jnp.max(logits, axis=1, keepdims=True)
    a1 = jnp.min(jnp.where(logits == m1, lane, E), axis=1, keepdims=True)
    masked = jnp.where(lane == a1, -jnp.inf, logits)
    m2 = jnp.max(masked, axis=1, keepdims=True)
    a2 = jnp.min(jnp.where(masked == m2, lane, E), axis=1, keepdims=True)

    w2 = jax.nn.sigmoid(m2 - m1)
    w1_ref[...] = 1.0 - w2
    w2_ref[...] = w2

    z = jnp.exp(logits - m1)
    p = z / jnp.sum(z, axis=1, keepdims=True)
    p_mean = jnp.sum(p, axis=0, keepdims=True) / N

    oh1 = (lane == a1).astype(jnp.float32)
    oh2 = (lane == a2).astype(jnp.float32)
    r_i = lax.broadcasted_iota(jnp.int32, (CB, CB), 0)
    c_i = lax.broadcasted_iota(jnp.int32, (CB, CB), 1)
    tri = (r_i >= c_i).astype(jnp.float32)

    def cumsum_rows(oh):
        blocks = []
        carry = jnp.zeros((1, E), jnp.float32)
        for b in range(N // CB):
            cs = lax.dot_general(tri, oh[b * CB:(b + 1) * CB, :],
                                 (((1,), (0,)), ((), ())),
                                 preferred_element_type=jnp.float32,
                                 precision=_HIGH) + carry
            carry = cs[CB - 1:CB, :]
            blocks.append(cs)
        return jnp.concatenate(blocks, axis=0), carry

    cum1, c1 = cumsum_rows(oh1)
    cum2, c2 = cumsum_rows(oh2)
    c = c1 + c2

    aux_ref[...] = jnp.sum(c / (N * K) * p_mean, axis=1, keepdims=True) * E

    tiles_e = jnp.floor((c + (TM - 1)) / TM)
    t8r = lax.broadcasted_iota(jnp.int32, (E, E), 0)
    t8c = lax.broadcasted_iota(jnp.int32, (E, E), 1)
    tri8 = (t8r <= t8c).astype(jnp.float32)
    padded_end = lax.dot_general(tiles_e, tri8, (((1,), (0,)), ((), ())),
                                 preferred_element_type=jnp.float32,
                                 precision=_HIGH)
    padded_off = (padded_end - tiles_e) * TM

    dest1 = jnp.sum(oh1 * (padded_off + cum1 - 1.0), axis=1, keepdims=True)
    dest2 = jnp.sum(oh2 * (padded_off + c1 + cum2 - 1.0), axis=1,
                    keepdims=True)
    dest1_ref[...] = dest1.astype(jnp.int32)
    dest2_ref[...] = dest2.astype(jnp.int32)

    t_col = lax.broadcasted_iota(jnp.int32, (NT, E), 0).astype(jnp.float32)
    te = jnp.sum((t_col >= padded_end).astype(jnp.float32), axis=1,
                 keepdims=True)
    te_ref[...] = te.astype(jnp.int32)


def _run_router(logits):
    return pl.pallas_call(
        _router_kernel,
        out_shape=(
            jax.ShapeDtypeStruct((N, 1), jnp.int32),
            jax.ShapeDtypeStruct((N, 1), jnp.int32),
            jax.ShapeDtypeStruct((N, 1), jnp.float32),
            jax.ShapeDtypeStruct((N, 1), jnp.float32),
            jax.ShapeDtypeStruct((1, 1), jnp.float32),
            jax.ShapeDtypeStruct((NT, 1), jnp.int32),
        ),
    )(logits)


NW = 32
TPW = N // NW
CH = 32


@functools.lru_cache(maxsize=None)
def _make_dispatch():
    mesh = plsc.VectorSubcoreMesh(core_axis_name="c", subcore_axis_name="s")

    @functools.partial(
        pl.kernel, mesh=mesh,
        out_type=(
            jax.ShapeDtypeStruct((PTOT, D), jnp.float32),
            jax.ShapeDtypeStruct((PTOT,), jnp.float32),
        ),
        scratch_types=[
            pltpu.VMEM((2, CH), jnp.int32),
            pltpu.VMEM((2, CH), jnp.float32),
            pltpu.VMEM((CH, D), jnp.float32),
            pltpu.SemaphoreType.DMA,
        ],
    )
    def dispatch(x_hbm, d1_hbm, d2_hbm, w1_hbm, w2_hbm, sx_hbm, sw_hbm,
                 idx_v, w_v, rows_v, sem):
        wid = lax.axis_index("s") * 2 + lax.axis_index("c")
        for ch in range(TPW // CH):
            b = wid * TPW + ch * CH
            pltpu.sync_copy(d1_hbm.at[pl.ds(b, CH)], idx_v.at[0])
            pltpu.sync_copy(d2_hbm.at[pl.ds(b, CH)], idx_v.at[1])
            pltpu.sync_copy(w1_hbm.at[pl.ds(b, CH)], w_v.at[0])
            pltpu.sync_copy(w2_hbm.at[pl.ds(b, CH)], w_v.at[1])
            pltpu.sync_copy(x_hbm.at[pl.ds(b, CH)], rows_v)
            cps = [
                pltpu.async_copy(rows_v, sx_hbm.at[idx_v.at[0]], sem),
                pltpu.async_copy(rows_v, sx_hbm.at[idx_v.at[1]], sem),
                pltpu.async_copy(w_v.at[0], sw_hbm.at[idx_v.at[0]], sem),
                pltpu.async_copy(w_v.at[1], sw_hbm.at[idx_v.at[1]], sem),
            ]
            for cp in cps:
                cp.wait()
            pltpu.sync_copy(rows_v, sx_hbm.at[pl.ds(SHBASE + b, CH)])

    return dispatch


def _dispatch(flat, d1, d2, w1, w2):
    return _make_dispatch()(flat, d1, d2, w1, w2)


def _group_kernel(te_ref, xs_ref, sw_ref, gw_ref, uw_ref, dw_ref, sg_ref,
                  su_ref, sd_ref, out_ref):
    t = pl.program_id(0)
    h = pl.program_id(1)
    is_sh = te_ref[t] == E

    xb = xs_ref[...].astype(jnp.bfloat16)
    gw = jnp.where(is_sh, sg_ref[0], gw_ref[0]).astype(jnp.bfloat16)
    uw = jnp.where(is_sh, su_ref[0], uw_ref[0]).astype(jnp.bfloat16)
    dw = jnp.where(is_sh, sd_ref[0], dw_ref[0]).astype(jnp.bfloat16)

    g = lax.dot_general(xb, gw, (((1,), (1,)), ((), ())),
                        preferred_element_type=jnp.float32)
    u = lax.dot_general(xb, uw, (((1,), (1,)), ((), ())),
                        preferred_element_type=jnp.float32)
    hact = (g * jax.nn.sigmoid(g) * u).astype(jnp.bfloat16)
    contrib = lax.dot_general(hact, dw, (((1,), (1,)), ((), ())),
                              preferred_element_type=jnp.float32)
    sw = jnp.where(is_sh, 1.0, sw_ref[...])
    contrib = contrib * sw

    @pl.when(h == 0)
    def _():
        out_ref[...] = contrib

    @pl.when(h != 0)
    def _():
        out_ref[...] += contrib


def _run_group(te, sorted_x, sorted_w, gate_w, up_w, down_w, sh_gate_w,
               sh_up_w, sh_down_w):
    def emap(t, h, te_s):
        return (jnp.minimum(te_s[t], E - 1), h, 0)

    def dmap(t, h, te_s):
        return (jnp.minimum(te_s[t], E - 1), 0, h)

    grid_spec = pltpu.PrefetchScalarGridSpec(
        num_scalar_prefetch=1,
        grid=(NT, H // HB),
        in_specs=[
            pl.BlockSpec((TM, D), lambda t, h, te_s: (t, 0)),
            pl.BlockSpec((TM, 1), lambda t, h, te_s: (t, 0)),
            pl.BlockSpec((1, HB, D), emap),
            pl.BlockSpec((1, HB, D), emap),
            pl.BlockSpec((1, D, HB), dmap),
            pl.BlockSpec((1, HB, D), lambda t, h, te_s: (0, h, 0)),
            pl.BlockSpec((1, HB, D), lambda t, h, te_s: (0, h, 0)),
            pl.BlockSpec((1, D, HB), lambda t, h, te_s: (0, 0, h)),
        ],
        out_specs=pl.BlockSpec((TM, D), lambda t, h, te_s: (t, 0)),
    )
    return pl.pallas_call(
        _group_kernel,
        grid_spec=grid_spec,
        out_shape=jax.ShapeDtypeStruct((PTOT, D), jnp.float32),
    )(te, sorted_x, sorted_w, gate_w, up_w, down_w, sh_gate_w, sh_up_w,
      sh_down_w)


CC = 16


@functools.lru_cache(maxsize=None)
def _make_combine():
    mesh = plsc.VectorSubcoreMesh(core_axis_name="c", subcore_axis_name="s")

    @functools.partial(
        pl.kernel, mesh=mesh,
        out_type=jax.ShapeDtypeStruct((N, D), jnp.float32),
        scratch_types=[
            pltpu.VMEM((2, CC), jnp.int32),
            pltpu.VMEM((CC, D), jnp.float32),
            pltpu.VMEM((CC, D), jnp.float32),
            pltpu.VMEM((CC, D), jnp.float32),
            pltpu.SemaphoreType.DMA,
        ],
    )
    def combine(so_hbm, d1_hbm, d2_hbm, out_hbm,
                idx_v, r1_v, r2_v, sh_v, sem):
        wid = lax.axis_index("s") * 2 + lax.axis_index("c")
        base = wid * TPW
        for ch in range(TPW // CC):
            b = base + ch * CC
            pltpu.sync_copy(d1_hbm.at[pl.ds(b, CC)], idx_v.at[0])
            pltpu.sync_copy(d2_hbm.at[pl.ds(b, CC)], idx_v.at[1])
            g1 = pltpu.async_copy(so_hbm.at[idx_v.at[0]], r1_v, sem)
            g2 = pltpu.async_copy(so_hbm.at[idx_v.at[1]], r2_v, sem)
            g3 = pltpu.async_copy(so_hbm.at[pl.ds(SHBASE + b, CC)], sh_v,
                                  sem)
            g1.wait()
            g2.wait()
            g3.wait()
            for j in range(CC):
                def body(ci, carry):
                    sl = pl.ds(ci * 16, 16)
                    sh_v[j, sl] = (r1_v[j, sl] + r2_v[j, sl] + sh_v[j, sl])
                    return carry

                lax.fori_loop(0, D // 16, body, 0)
            pltpu.sync_copy(sh_v, out_hbm.at[pl.ds(b, CC)])

    return combine


def _combine(sorted_out, d1, d2):
    return _make_combine()(sorted_out, d1, d2)


def kernel(x, router_w, gate_w, up_w, down_w, sh_gate_w, sh_up_w, sh_down_w):
    flat = x.reshape(N, D)
    logits = flat @ router_w.T
    dest1, dest2, w1, w2, aux, te = _run_router(logits)
    d1 = dest1.reshape(N)
    d2 = dest2.reshape(N)
    sorted_x, sorted_w = _dispatch(flat, d1, d2, w1.reshape(N),
                                   w2.reshape(N))
    sorted_out = _run_group(te.reshape(NT), sorted_x,
                            sorted_w.reshape(PTOT, 1), gate_w, up_w, down_w,
                            sh_gate_w, sh_up_w, sh_down_w)
    out = _combine(sorted_out, d1, d2)
    return out.reshape(1, N, D), aux.reshape(())


# speedup vs baseline: 2.1349x; 2.1349x over previous
import functools

import jax
import jax.numpy as jnp
from jax import lax
from jax.experimental import pallas as pl
from jax.experimental.pallas import tpu as pltpu
from jax.experimental.pallas import tpu_sc as plsc

D = 2048
E = 8
K = 2
H = 1024
N = 2048
TM = 256
NT = N * K // TM + E
PTOT = NT * TM
CB = 256
SH_NB = 4
SH_HB = 512

_HIGH = jax.lax.Precision.HIGHEST


def _router_kernel(lg_ref, dest1_ref, dest2_ref, w1_ref, w2_ref,
                   aux_ref, te_ref):
    logits = lg_ref[...]

    lane = lax.broadcasted_iota(jnp.int32, (N, E), 1)
    m1 = jnp.max(logits, axis=1, keepdims=True)
    a1 = jnp.min(jnp.where(logits == m1, lane, E), axis=1, keepdims=True)
    masked = jnp.where(lane == a1, -jnp.inf, logits)
    m2 = jnp.max(masked, axis=1, keepdims=True)
    a2 = jnp.min(jnp.where(masked == m2, lane, E), axis=1, keepdims=True)

    w2 = jax.nn.sigmoid(m2 - m1)
    w1_ref[...] = 1.0 - w2
    w2_ref[...] = w2

    z = jnp.exp(logits - m1)
    p = z / jnp.sum(z, axis=1, keepdims=True)
    p_mean = jnp.sum(p, axis=0, keepdims=True) / N

    oh1 = (lane == a1).astype(jnp.float32)
    oh2 = (lane == a2).astype(jnp.float32)
    r_i = lax.broadcasted_iota(jnp.int32, (CB, CB), 0)
    c_i = lax.broadcasted_iota(jnp.int32, (CB, CB), 1)
    tri = (r_i >= c_i).astype(jnp.float32)

    def cumsum_rows(oh):
        blocks = []
        carry = jnp.zeros((1, E), jnp.float32)
        for b in range(N // CB):
            cs = lax.dot_general(tri, oh[b * CB:(b + 1) * CB, :],
                                 (((1,), (0,)), ((), ())),
                                 preferred_element_type=jnp.float32,
                                 precision=_HIGH) + carry
            carry = cs[CB - 1:CB, :]
            blocks.append(cs)
        return jnp.concatenate(blocks, axis=0), carry

    cum1, c1 = cumsum_rows(oh1)
    cum2, c2 = cumsum_rows(oh2)
    c = c1 + c2

    aux_ref[...] = jnp.sum(c / (N * K) * p_mean, axis=1, keepdims=True) * E

    tiles_e = jnp.floor((c + (TM - 1)) / TM)
    t8r = lax.broadcasted_iota(jnp.int32, (E, E), 0)
    t8c = lax.broadcasted_iota(jnp.int32, (E, E), 1)
    tri8 = (t8r <= t8c).astype(jnp.float32)
    padded_end = lax.dot_general(tiles_e, tri8, (((1,), (0,)), ((), ())),
                                 preferred_element_type=jnp.float32,
                                 precision=_HIGH)
    padded_off = (padded_end - tiles_e) * TM

    dest1 = jnp.sum(oh1 * (padded_off + cum1 - 1.0), axis=1, keepdims=True)
    dest2 = jnp.sum(oh2 * (padded_off + c1 + cum2 - 1.0), axis=1,
                    keepdims=True)
    dest1_ref[...] = dest1.astype(jnp.int32)
    dest2_ref[...] = dest2.astype(jnp.int32)

    t_col = lax.broadcasted_iota(jnp.int32, (NT, E), 0).astype(jnp.float32)
    te = jnp.sum((t_col >= padded_end).astype(jnp.float32), axis=1,
                 keepdims=True)
    te_ref[...] = te.astype(jnp.int32)


def _run_router(logits):
    return pl.pallas_call(
        _router_kernel,
        out_shape=(
            jax.ShapeDtypeStruct((N, 1), jnp.int32),
            jax.ShapeDtypeStruct((N, 1), jnp.int32),
            jax.ShapeDtypeStruct((N, 1), jnp.float32),
            jax.ShapeDtypeStruct((N, 1), jnp.float32),
            jax.ShapeDtypeStruct((1, 1), jnp.float32),
            jax.ShapeDtypeStruct((NT, 1), jnp.int32),
        ),
    )(logits)


NW = 32
TPW = N // NW
CH = 32


@functools.lru_cache(maxsize=None)
def _make_dispatch():
    mesh = plsc.VectorSubcoreMesh(core_axis_name="c", subcore_axis_name="s")

    @functools.partial(
        pl.kernel, mesh=mesh,
        out_type=(
            jax.ShapeDtypeStruct((PTOT, D), jnp.float32),
            jax.ShapeDtypeStruct((PTOT,), jnp.float32),
        ),
        scratch_types=[
            pltpu.VMEM((2, CH), jnp.int32),
            pltpu.VMEM((2, CH), jnp.float32),
            pltpu.VMEM((CH, D), jnp.float32),
            pltpu.SemaphoreType.DMA,
        ],
    )
    def dispatch(x_hbm, d1_hbm, d2_hbm, w1_hbm, w2_hbm, sx_hbm, sw_hbm,
                 idx_v, w_v, rows_v, sem):
        wid = lax.axis_index("s") * 2 + lax.axis_index("c")
        for ch in range(TPW // CH):
            b = wid * TPW + ch * CH
            pltpu.sync_copy(d1_hbm.at[pl.ds(b, CH)], idx_v.at[0])
            pltpu.sync_copy(d2_hbm.at[pl.ds(b, CH)], idx_v.at[1])
            pltpu.sync_copy(w1_hbm.at[pl.ds(b, CH)], w_v.at[0])
            pltpu.sync_copy(w2_hbm.at[pl.ds(b, CH)], w_v.at[1])
            pltpu.sync_copy(x_hbm.at[pl.ds(b, CH)], rows_v)
            cps = [
                pltpu.async_copy(rows_v, sx_hbm.at[idx_v.at[0]], sem),
                pltpu.async_copy(rows_v, sx_hbm.at[idx_v.at[1]], sem),
                pltpu.async_copy(w_v.at[0], sw_hbm.at[idx_v.at[0]], sem),
                pltpu.async_copy(w_v.at[1], sw_hbm.at[idx_v.at[1]], sem),
            ]
            for cp in cps:
                cp.wait()

    return dispatch


def _dispatch(xb, d1, d2, w1, w2):
    return _make_dispatch()(xb, d1, d2, w1, w2)


def _group_kernel(te_ref, xs_ref, sw_ref, gw_ref, uw_ref, dw_ref, out_ref):
    t = pl.program_id(0)

    @pl.when(te_ref[t] < E)
    def _():
        xb = xs_ref[...].astype(jnp.bfloat16)
        gw = gw_ref[0].astype(jnp.bfloat16)
        uw = uw_ref[0].astype(jnp.bfloat16)
        dw = dw_ref[0].astype(jnp.bfloat16)
        g = lax.dot_general(xb, gw, (((1,), (1,)), ((), ())),
                            preferred_element_type=jnp.float32)
        u = lax.dot_general(xb, uw, (((1,), (1,)), ((), ())),
                            preferred_element_type=jnp.float32)
        hact = (g * jax.nn.sigmoid(g) * u).astype(jnp.bfloat16)
        contrib = lax.dot_general(hact, dw, (((1,), (1,)), ((), ())),
                                  preferred_element_type=jnp.float32)
        out_ref[...] = contrib * sw_ref[...]


def _run_group(te, sorted_x, sorted_w, gate_w, up_w, down_w):
    def emap(t, te_s):
        return (jnp.minimum(te_s[t], E - 1), 0, 0)

    grid_spec = pltpu.PrefetchScalarGridSpec(
        num_scalar_prefetch=1,
        grid=(NT,),
        in_specs=[
            pl.BlockSpec((TM, D), lambda t, te_s: (t, 0)),
            pl.BlockSpec((TM, 1), lambda t, te_s: (t, 0)),
            pl.BlockSpec((1, H, D), emap),
            pl.BlockSpec((1, H, D), emap),
            pl.BlockSpec((1, D, H), emap),
        ],
        out_specs=pl.BlockSpec((TM, D), lambda t, te_s: (t, 0)),
    )
    return pl.pallas_call(
        _group_kernel,
        grid_spec=grid_spec,
        out_shape=jax.ShapeDtypeStruct((PTOT, D), jnp.float32),
        compiler_params=pltpu.CompilerParams(
            vmem_limit_bytes=67108864),
    )(te, sorted_x, sorted_w, gate_w, up_w, down_w)


def _shared_kernel(x_ref, gw_ref, uw_ref, dw_ref, out_ref):
    h = pl.program_id(1)
    xb = x_ref[...].astype(jnp.bfloat16)
    gw = gw_ref[0].astype(jnp.bfloat16)
    uw = uw_ref[0].astype(jnp.bfloat16)
    dw = dw_ref[0].astype(jnp.bfloat16)
    g = lax.dot_general(xb, gw, (((1,), (1,)), ((), ())),
                        preferred_element_type=jnp.float32)
    u = lax.dot_general(xb, uw, (((1,), (1,)), ((), ())),
                        preferred_element_type=jnp.float32)
    hact = (g * jax.nn.sigmoid(g) * u).astype(jnp.bfloat16)
    contrib = lax.dot_general(hact, dw, (((1,), (1,)), ((), ())),
                              preferred_element_type=jnp.float32)

    @pl.when(h == 0)
    def _():
        out_ref[...] = contrib

    @pl.when(h != 0)
    def _():
        out_ref[...] += contrib


def _run_shared(flat, sh_gate_w, sh_up_w, sh_down_w):
    nb = N // SH_NB
    return pl.pallas_call(
        _shared_kernel,
        grid=(SH_NB, H // SH_HB),
        in_specs=[
            pl.BlockSpec((nb, D), lambda n, h: (n, 0)),
            pl.BlockSpec((1, SH_HB, D), lambda n, h: (0, h, 0)),
            pl.BlockSpec((1, SH_HB, D), lambda n, h: (0, h, 0)),
            pl.BlockSpec((1, D, SH_HB), lambda n, h: (0, 0, h)),
        ],
        out_specs=pl.BlockSpec((nb, D), lambda n, h: (n, 0)),
        out_shape=jax.ShapeDtypeStruct((N, D), jnp.float32),
    )(flat, sh_gate_w, sh_up_w, sh_down_w)


CC = 16


@functools.lru_cache(maxsize=None)
def _make_combine():
    mesh = plsc.VectorSubcoreMesh(core_axis_name="c", subcore_axis_name="s")

    @functools.partial(
        pl.kernel, mesh=mesh,
        out_type=jax.ShapeDtypeStruct((N, D), jnp.float32),
        scratch_types=[
            pltpu.VMEM((2, CC), jnp.int32),
            pltpu.VMEM((CC, D), jnp.float32),
            pltpu.VMEM((CC, D), jnp.float32),
            pltpu.VMEM((CC, D), jnp.float32),
            pltpu.SemaphoreType.DMA,
        ],
    )
    def combine(so_hbm, sh_hbm, d1_hbm, d2_hbm, out_hbm,
                idx_v, r1_v, r2_v, sh_v, sem):
        wid = lax.axis_index("s") * 2 + lax.axis_index("c")
        base = wid * TPW
        for ch in range(TPW // CC):
            b = base + ch * CC
            pltpu.sync_copy(d1_hbm.at[pl.ds(b, CC)], idx_v.at[0])
            pltpu.sync_copy(d2_hbm.at[pl.ds(b, CC)], idx_v.at[1])
            g1 = pltpu.async_copy(so_hbm.at[idx_v.at[0]], r1_v, sem)
            g2 = pltpu.async_copy(so_hbm.at[idx_v.at[1]], r2_v, sem)
            g3 = pltpu.async_copy(sh_hbm.at[pl.ds(b, CC)], sh_v, sem)
            g1.wait()
            g2.wait()
            g3.wait()
            for j in range(CC):
                def body(ci, carry):
                    sl = pl.ds(ci * 16, 16)
                    sh_v[j, sl] = (r1_v[j, sl] + r2_v[j, sl] + sh_v[j, sl])
                    return carry

                lax.fori_loop(0, D // 16, body, 0)
            pltpu.sync_copy(sh_v, out_hbm.at[pl.ds(b, CC)])

    return combine


def _combine(sorted_out, shared_out, d1, d2):
    return _make_combine()(sorted_out, shared_out, d1, d2)


def kernel(x, router_w, gate_w, up_w, down_w, sh_gate_w, sh_up_w, sh_down_w):
    flat = x.reshape(N, D)
    logits = flat @ router_w.T
    dest1, dest2, w1, w2, aux, te = _run_router(logits)
    d1 = dest1.reshape(N)
    d2 = dest2.reshape(N)
    sorted_x, sorted_w = _dispatch(flat, d1, d2, w1.reshape(N),
                                   w2.reshape(N))
    sorted_out = _run_group(te.reshape(NT), sorted_x,
                            sorted_w.reshape(PTOT, 1), gate_w, up_w, down_w)
    shared_out = _run_shared(flat, sh_gate_w, sh_up_w, sh_down_w)
    out = _combine(sorted_out, shared_out, d1, d2)
    return out.reshape(1, N, D), aux.reshape(())


# speedup vs baseline: 2.1498x; 1.0070x over previous
import functools

import jax
import jax.numpy as jnp
from jax import lax
from jax.experimental import pallas as pl
from jax.experimental.pallas import tpu as pltpu
from jax.experimental.pallas import tpu_sc as plsc

D = 2048
E = 8
K = 2
H = 1024
N = 2048
TM = 256
NT = N * K // TM + E
PTOT = NT * TM
CB = 256
SH_NB = 4
SH_HB = 512

_HIGH = jax.lax.Precision.HIGHEST


def _router_kernel(lg_ref, dest1_ref, dest2_ref, w1_ref, w2_ref,
                   aux_ref, te_ref):
    logits = lg_ref[...]

    lane = lax.broadcasted_iota(jnp.int32, (N, E), 1)
    m1 = jnp.max(logits, axis=1, keepdims=True)
    a1 = jnp.min(jnp.where(logits == m1, lane, E), axis=1, keepdims=True)
    masked = jnp.where(lane == a1, -jnp.inf, logits)
    m2 = jnp.max(masked, axis=1, keepdims=True)
    a2 = jnp.min(jnp.where(masked == m2, lane, E), axis=1, keepdims=True)

    w2 = jax.nn.sigmoid(m2 - m1)
    w1_ref[...] = 1.0 - w2
    w2_ref[...] = w2

    z = jnp.exp(logits - m1)
    p = z / jnp.sum(z, axis=1, keepdims=True)
    p_mean = jnp.sum(p, axis=0, keepdims=True) / N

    oh1 = (lane == a1).astype(jnp.float32)
    oh2 = (lane == a2).astype(jnp.float32)
    r_i = lax.broadcasted_iota(jnp.int32, (CB, CB), 0)
    c_i = lax.broadcasted_iota(jnp.int32, (CB, CB), 1)
    tri = (r_i >= c_i).astype(jnp.float32)

    def cumsum_rows(oh):
        blocks = []
        carry = jnp.zeros((1, E), jnp.float32)
        for b in range(N // CB):
            cs = lax.dot_general(tri, oh[b * CB:(b + 1) * CB, :],
                                 (((1,), (0,)), ((), ())),
                                 preferred_element_type=jnp.float32,
                                 precision=_HIGH) + carry
            carry = cs[CB - 1:CB, :]
            blocks.append(cs)
        return jnp.concatenate(blocks, axis=0), carry

    cum1, c1 = cumsum_rows(oh1)
    cum2, c2 = cumsum_rows(oh2)
    c = c1 + c2

    aux_ref[...] = jnp.sum(c / (N * K) * p_mean, axis=1, keepdims=True) * E

    tiles_e = jnp.floor((c + (TM - 1)) / TM)
    t8r = lax.broadcasted_iota(jnp.int32, (E, E), 0)
    t8c = lax.broadcasted_iota(jnp.int32, (E, E), 1)
    tri8 = (t8r <= t8c).astype(jnp.float32)
    padded_end = lax.dot_general(tiles_e, tri8, (((1,), (0,)), ((), ())),
                                 preferred_element_type=jnp.float32,
                                 precision=_HIGH)
    padded_off = (padded_end - tiles_e) * TM

    dest1 = jnp.sum(oh1 * (padded_off + cum1 - 1.0), axis=1, keepdims=True)
    dest2 = jnp.sum(oh2 * (padded_off + c1 + cum2 - 1.0), axis=1,
                    keepdims=True)
    dest1_ref[...] = dest1.astype(jnp.int32)
    dest2_ref[...] = dest2.astype(jnp.int32)

    t_col = lax.broadcasted_iota(jnp.int32, (NT, E), 0).astype(jnp.float32)
    te = jnp.sum((t_col >= padded_end).astype(jnp.float32), axis=1,
                 keepdims=True)
    te_ref[...] = te.astype(jnp.int32)


def _run_router(logits):
    return pl.pallas_call(
        _router_kernel,
        out_shape=(
            jax.ShapeDtypeStruct((N, 1), jnp.int32),
            jax.ShapeDtypeStruct((N, 1), jnp.int32),
            jax.ShapeDtypeStruct((N, 1), jnp.float32),
            jax.ShapeDtypeStruct((N, 1), jnp.float32),
            jax.ShapeDtypeStruct((1, 1), jnp.float32),
            jax.ShapeDtypeStruct((NT, 1), jnp.int32),
        ),
    )(logits)


NW = 32
TPW = N // NW
CH = 32


@functools.lru_cache(maxsize=None)
def _make_dispatch():
    mesh = plsc.VectorSubcoreMesh(core_axis_name="c", subcore_axis_name="s")

    @functools.partial(
        pl.kernel, mesh=mesh,
        out_type=(
            jax.ShapeDtypeStruct((PTOT, D), jnp.float32),
            jax.ShapeDtypeStruct((PTOT,), jnp.float32),
        ),
        scratch_types=[
            pltpu.VMEM((2, CH), jnp.int32),
            pltpu.VMEM((2, CH), jnp.float32),
            pltpu.VMEM((CH, D), jnp.float32),
            pltpu.SemaphoreType.DMA,
        ],
    )
    def dispatch(x_hbm, d1_hbm, d2_hbm, w1_hbm, w2_hbm, sx_hbm, sw_hbm,
                 idx_v, w_v, rows_v, sem):
        wid = lax.axis_index("s") * 2 + lax.axis_index("c")
        for ch in range(TPW // CH):
            b = wid * TPW + ch * CH
            lds = [
                pltpu.async_copy(d1_hbm.at[pl.ds(b, CH)], idx_v.at[0], sem),
                pltpu.async_copy(d2_hbm.at[pl.ds(b, CH)], idx_v.at[1], sem),
                pltpu.async_copy(w1_hbm.at[pl.ds(b, CH)], w_v.at[0], sem),
                pltpu.async_copy(w2_hbm.at[pl.ds(b, CH)], w_v.at[1], sem),
                pltpu.async_copy(x_hbm.at[pl.ds(b, CH)], rows_v, sem),
            ]
            for cp in lds:
                cp.wait()
            cps = [
                pltpu.async_copy(rows_v, sx_hbm.at[idx_v.at[0]], sem),
                pltpu.async_copy(rows_v, sx_hbm.at[idx_v.at[1]], sem),
                pltpu.async_copy(w_v.at[0], sw_hbm.at[idx_v.at[0]], sem),
                pltpu.async_copy(w_v.at[1], sw_hbm.at[idx_v.at[1]], sem),
            ]
            for cp in cps:
                cp.wait()

    return dispatch


def _dispatch(xb, d1, d2, w1, w2):
    return _make_dispatch()(xb, d1, d2, w1, w2)


def _group_kernel(te_ref, xs_ref, sw_ref, gw_ref, uw_ref, dw_ref, out_ref):
    t = pl.program_id(0)

    @pl.when(te_ref[t] < E)
    def _():
        xb = xs_ref[...].astype(jnp.bfloat16)
        gw = gw_ref[0].astype(jnp.bfloat16)
        uw = uw_ref[0].astype(jnp.bfloat16)
        dw = dw_ref[0].astype(jnp.bfloat16)
        g = lax.dot_general(xb, gw, (((1,), (1,)), ((), ())),
                            preferred_element_type=jnp.float32)
        u = lax.dot_general(xb, uw, (((1,), (1,)), ((), ())),
                            preferred_element_type=jnp.float32)
        hact = (g * jax.nn.sigmoid(g) * u).astype(jnp.bfloat16)
        contrib = lax.dot_general(hact, dw, (((1,), (1,)), ((), ())),
                                  preferred_element_type=jnp.float32)
        out_ref[...] = contrib * sw_ref[...]


def _run_group(te, sorted_x, sorted_w, gate_w, up_w, down_w):
    def emap(t, te_s):
        return (jnp.minimum(te_s[t], E - 1), 0, 0)

    grid_spec = pltpu.PrefetchScalarGridSpec(
        num_scalar_prefetch=1,
        grid=(NT,),
        in_specs=[
            pl.BlockSpec((TM, D), lambda t, te_s: (t, 0)),
            pl.BlockSpec((TM, 1), lambda t, te_s: (t, 0)),
            pl.BlockSpec((1, H, D), emap),
            pl.BlockSpec((1, H, D), emap),
            pl.BlockSpec((1, D, H), emap),
        ],
        out_specs=pl.BlockSpec((TM, D), lambda t, te_s: (t, 0)),
    )
    return pl.pallas_call(
        _group_kernel,
        grid_spec=grid_spec,
        out_shape=jax.ShapeDtypeStruct((PTOT, D), jnp.float32),
        compiler_params=pltpu.CompilerParams(
            vmem_limit_bytes=67108864),
    )(te, sorted_x, sorted_w, gate_w, up_w, down_w)


def _shared_kernel(x_ref, gw_ref, uw_ref, dw_ref, out_ref):
    h = pl.program_id(1)
    xb = x_ref[...].astype(jnp.bfloat16)
    gw = gw_ref[0].astype(jnp.bfloat16)
    uw = uw_ref[0].astype(jnp.bfloat16)
    dw = dw_ref[0].astype(jnp.bfloat16)
    g = lax.dot_general(xb, gw, (((1,), (1,)), ((), ())),
                        preferred_element_type=jnp.float32)
    u = lax.dot_general(xb, uw, (((1,), (1,)), ((), ())),
                        preferred_element_type=jnp.float32)
    hact = (g * jax.nn.sigmoid(g) * u).astype(jnp.bfloat16)
    contrib = lax.dot_general(hact, dw, (((1,), (1,)), ((), ())),
                              preferred_element_type=jnp.float32)

    @pl.when(h == 0)
    def _():
        out_ref[...] = contrib

    @pl.when(h != 0)
    def _():
        out_ref[...] += contrib


def _run_shared(flat, sh_gate_w, sh_up_w, sh_down_w):
    nb = N // SH_NB
    return pl.pallas_call(
        _shared_kernel,
        grid=(SH_NB, H // SH_HB),
        in_specs=[
            pl.BlockSpec((nb, D), lambda n, h: (n, 0)),
            pl.BlockSpec((1, SH_HB, D), lambda n, h: (0, h, 0)),
            pl.BlockSpec((1, SH_HB, D), lambda n, h: (0, h, 0)),
            pl.BlockSpec((1, D, SH_HB), lambda n, h: (0, 0, h)),
        ],
        out_specs=pl.BlockSpec((nb, D), lambda n, h: (n, 0)),
        out_shape=jax.ShapeDtypeStruct((N, D), jnp.float32),
    )(flat, sh_gate_w, sh_up_w, sh_down_w)


CC = 16


@functools.lru_cache(maxsize=None)
def _make_combine():
    mesh = plsc.VectorSubcoreMesh(core_axis_name="c", subcore_axis_name="s")

    @functools.partial(
        pl.kernel, mesh=mesh,
        out_type=jax.ShapeDtypeStruct((N, D), jnp.float32),
        scratch_types=[
            pltpu.VMEM((2, CC), jnp.int32),
            pltpu.VMEM((CC, D), jnp.float32),
            pltpu.VMEM((CC, D), jnp.float32),
            pltpu.VMEM((CC, D), jnp.float32),
            pltpu.SemaphoreType.DMA,
        ],
    )
    def combine(so_hbm, sh_hbm, d1_hbm, d2_hbm, out_hbm,
                idx_v, r1_v, r2_v, sh_v, sem):
        wid = lax.axis_index("s") * 2 + lax.axis_index("c")
        base = wid * TPW
        for ch in range(TPW // CC):
            b = base + ch * CC
            pltpu.sync_copy(d1_hbm.at[pl.ds(b, CC)], idx_v.at[0])
            pltpu.sync_copy(d2_hbm.at[pl.ds(b, CC)], idx_v.at[1])
            g1 = pltpu.async_copy(so_hbm.at[idx_v.at[0]], r1_v, sem)
            g2 = pltpu.async_copy(so_hbm.at[idx_v.at[1]], r2_v, sem)
            g3 = pltpu.async_copy(sh_hbm.at[pl.ds(b, CC)], sh_v, sem)
            g1.wait()
            g2.wait()
            g3.wait()
            for j in range(CC):
                def body(ci, carry):
                    for k in range(4):
                        sl = pl.ds(ci * 64 + k * 16, 16)
                        sh_v[j, sl] = (r1_v[j, sl] + r2_v[j, sl]
                                       + sh_v[j, sl])
                    return carry

                lax.fori_loop(0, D // 64, body, 0)
            pltpu.sync_copy(sh_v, out_hbm.at[pl.ds(b, CC)])

    return combine


def _combine(sorted_out, shared_out, d1, d2):
    return _make_combine()(sorted_out, shared_out, d1, d2)


def kernel(x, router_w, gate_w, up_w, down_w, sh_gate_w, sh_up_w, sh_down_w):
    flat = x.reshape(N, D)
    logits = flat @ router_w.T
    dest1, dest2, w1, w2, aux, te = _run_router(logits)
    d1 = dest1.reshape(N)
    d2 = dest2.reshape(N)
    sorted_x, sorted_w = _dispatch(flat, d1, d2, w1.reshape(N),
                                   w2.reshape(N))
    sorted_out = _run_group(te.reshape(NT), sorted_x,
                            sorted_w.reshape(PTOT, 1), gate_w, up_w, down_w)
    shared_out = _run_shared(flat, sh_gate_w, sh_up_w, sh_down_w)
    out = _combine(sorted_out, shared_out, d1, d2)
    return out.reshape(1, N, D), aux.reshape(())
